# baseline (speedup 1.0000x reference)
import functools

import jax
import jax.numpy as jnp
from jax import lax
from jax.experimental import pallas as pl
from jax.experimental.pallas import tpu as pltpu
from jax.experimental.pallas import tpu_sc as plsc

CELL_SIDE = 0.5
N_GRID = 6
N_BINS = N_GRID * N_GRID
GRID8 = 8
N_PAD = GRID8 * GRID8
N_AGENTS = 4096
HIDDEN = 128

NUM_CORES = 2
NUM_SUBCORES = 16
NUM_WORKERS = NUM_CORES * NUM_SUBCORES
ROWS_PER_WORKER = N_AGENTS // NUM_WORKERS
LANES = 16
N_CHUNKS = N_AGENTS // LANES
SELF_CELL8 = (N_GRID // 2 + 1) * GRID8 + N_GRID // 2 + 1
REPLICAS = 8
HIST_W = REPLICAS * N_PAD


def _occupancy_sc(xs, ys):
  mesh = plsc.VectorSubcoreMesh(
      core_axis_name="c", subcore_axis_name="s",
      num_cores=NUM_CORES, num_subcores=NUM_SUBCORES)

  @functools.partial(
      pl.kernel,
      out_type=jax.ShapeDtypeStruct((N_AGENTS, N_PAD), jnp.float32),
      mesh=mesh,
      compiler_params=pltpu.CompilerParams(needs_layout_passes=False),
      scratch_types=[
          pltpu.VMEM((N_AGENTS,), jnp.float32),
          pltpu.VMEM((N_AGENTS,), jnp.float32),
          pltpu.VMEM((ROWS_PER_WORKER, HIST_W), jnp.float32),
          pltpu.VMEM((ROWS_PER_WORKER, N_PAD), jnp.float32),
      ],
  )
  def occ_kernel(xs_hbm, ys_hbm, out_hbm, xs_v, ys_v, hist_v, occ_v):
    wid = lax.axis_index("s") * NUM_CORES + lax.axis_index("c")
    base = wid * ROWS_PER_WORKER

    pltpu.sync_copy(xs_hbm, xs_v)
    pltpu.sync_copy(ys_hbm, ys_v)

    inv_cell = 1.0 / CELL_SIDE
    shift = float(N_GRID) / 2.0 + 1.0

    def scale_body(c, _):
      s = pl.ds(c * LANES, LANES)
      xs_v[s] = xs_v[s] * inv_cell
      ys_v[s] = ys_v[s] * inv_cell
      return 0
    lax.fori_loop(0, N_CHUNKS, scale_body, 0)

    zeros16 = jnp.zeros((LANES,), jnp.float32)
    ones16 = jnp.ones((LANES,), jnp.float32)
    top = float(GRID8 - 1)
    rep_off = (lax.iota(jnp.int32, LANES) & (REPLICAS - 1)) * N_PAD

    def agent_body(li, _):
      i_vec = jnp.full((LANES,), base + li, jnp.int32)
      ax = plsc.load_gather(xs_v, [i_vec]) - shift
      ay = plsc.load_gather(ys_v, [i_vec]) - shift
      li_vec = jnp.full((LANES,), li, jnp.int32)

      for k in range(HIST_W // LANES):
        hist_v[li, pl.ds(k * LANES, LANES)] = zeros16

      @plsc.parallel_loop(0, N_CHUNKS, unroll=4)
      def _(c):
        s = pl.ds(c * LANES, LANES)
        px = jnp.clip(xs_v[s] - ax, 0.0, top).astype(jnp.int32)
        py = jnp.clip(ys_v[s] - ay, 0.0, top).astype(jnp.int32)
        cell = px * GRID8 + py + rep_off
        plsc.addupdate_scatter(hist_v, [li_vec, cell], ones16)

      for k in range(N_PAD // LANES):
        acc = hist_v[li, pl.ds(k * LANES, LANES)]
        for r in range(1, REPLICAS):
          acc = acc + hist_v[li, pl.ds(r * N_PAD + k * LANES, LANES)]
        occ_v[li, pl.ds(k * LANES, LANES)] = acc

      return 0

    lax.fori_loop(0, ROWS_PER_WORKER, agent_body, 0)

    pltpu.sync_copy(occ_v, out_hbm.at[pl.ds(base, ROWS_PER_WORKER)])

  return occ_kernel(xs, ys)


def _linear_tc(occ, w64, b2):
  def body(occ_ref, wt_ref, b_ref, out_ref):
    acc = jnp.dot(occ_ref[...], wt_ref[...],
                  preferred_element_type=jnp.float32)
    out_ref[...] = acc + (b_ref[...] - wt_ref[SELF_CELL8:SELF_CELL8 + 1, :])

  return pl.pallas_call(
      body,
      out_shape=jax.ShapeDtypeStruct((N_AGENTS, HIDDEN), jnp.float32),
  )(occ, w64, b2)


@jax.jit
def kernel(hidden_in, cell_in, obs, W, b):
  del hidden_in, cell_in
  xs = obs[:, 0]
  ys = obs[:, 1]
  occ = _occupancy_sc(xs, ys)
  cells = jnp.arange(N_BINS)
  dest = (cells // N_GRID + 1) * GRID8 + cells % N_GRID + 1
  w64 = jnp.zeros((N_PAD, HIDDEN), jnp.float32).at[dest].set(W.T)
  b2 = b.reshape(1, HIDDEN)
  return _linear_tc(occ, w64, b2)


# speedup vs baseline: 1.1932x; 1.1932x over previous
import functools

import jax
import jax.numpy as jnp
from jax import lax
from jax.experimental import pallas as pl
from jax.experimental.pallas import tpu as pltpu
from jax.experimental.pallas import tpu_sc as plsc

CELL_SIDE = 0.5
N_GRID = 6
N_BINS = N_GRID * N_GRID
GRID8 = 8
N_PAD = GRID8 * GRID8
N_AGENTS = 4096
HIDDEN = 128

NUM_CORES = 2
NUM_SUBCORES = 16
NUM_WORKERS = NUM_CORES * NUM_SUBCORES
ROWS_PER_WORKER = N_AGENTS // NUM_WORKERS
LANES = 16
N_CHUNKS = N_AGENTS // LANES
HIST_WORDS = ROWS_PER_WORKER * N_PAD
SELF_CELL8 = (N_GRID // 2 + 1) * GRID8 + N_GRID // 2 + 1


def _occupancy_sc(xs, ys):
  mesh = plsc.VectorSubcoreMesh(
      core_axis_name="c", subcore_axis_name="s",
      num_cores=NUM_CORES, num_subcores=NUM_SUBCORES)

  @functools.partial(
      pl.kernel,
      out_type=jax.ShapeDtypeStruct((N_AGENTS * N_PAD,), jnp.float32),
      mesh=mesh,
      compiler_params=pltpu.CompilerParams(needs_layout_passes=False),
      scratch_types=[
          pltpu.VMEM((N_AGENTS,), jnp.float32),
          pltpu.VMEM((N_AGENTS,), jnp.float32),
          pltpu.VMEM((HIST_WORDS,), jnp.float32),
      ],
  )
  def occ_kernel(xs_hbm, ys_hbm, out_hbm, xs_v, ys_v, hist_v):
    wid = lax.axis_index("s") * NUM_CORES + lax.axis_index("c")
    base = wid * ROWS_PER_WORKER

    pltpu.sync_copy(xs_hbm, xs_v)
    pltpu.sync_copy(ys_hbm, ys_v)

    inv_cell = 1.0 / CELL_SIDE
    shift = float(N_GRID) / 2.0 + 1.0

    def scale_body(c, _):
      s = pl.ds(c * LANES, LANES)
      xs_v[s] = xs_v[s] * inv_cell
      ys_v[s] = ys_v[s] * inv_cell
      return 0
    lax.fori_loop(0, N_CHUNKS, scale_body, 0)

    zeros16 = jnp.zeros((LANES,), jnp.float32)
    ones16 = jnp.ones((LANES,), jnp.float32)
    top = float(GRID8 - 1)

    def agent_body(li, _):
      i_vec = jnp.full((LANES,), base + li, jnp.int32)
      ax = plsc.load_gather(xs_v, [i_vec]) - shift
      ay = plsc.load_gather(ys_v, [i_vec]) - shift
      row = li * N_PAD
      row_vec = jnp.full((LANES,), row, jnp.int32)

      for k in range(N_PAD // LANES):
        hist_v[pl.ds(row + k * LANES, LANES)] = zeros16

      @plsc.parallel_loop(0, N_CHUNKS, unroll=4)
      def _(c):
        s = pl.ds(c * LANES, LANES)
        px = jnp.clip(xs_v[s] - ax, 0.0, top).astype(jnp.int32)
        py = jnp.clip(ys_v[s] - ay, 0.0, top).astype(jnp.int32)
        addr = row_vec + px * GRID8 + py
        plsc.addupdate_scatter(hist_v, [addr], ones16)

      return 0

    lax.fori_loop(0, ROWS_PER_WORKER, agent_body, 0)

    pltpu.sync_copy(hist_v, out_hbm.at[pl.ds(base * N_PAD, HIST_WORDS)])

  return occ_kernel(xs, ys)


def _linear_tc(occ, w64, b2):
  def body(occ_ref, wt_ref, b_ref, out_ref):
    acc = jnp.dot(occ_ref[...], wt_ref[...],
                  preferred_element_type=jnp.float32)
    out_ref[...] = acc + (b_ref[...] - wt_ref[SELF_CELL8:SELF_CELL8 + 1, :])

  return pl.pallas_call(
      body,
      out_shape=jax.ShapeDtypeStruct((N_AGENTS, HIDDEN), jnp.float32),
  )(occ, w64, b2)


@jax.jit
def kernel(hidden_in, cell_in, obs, W, b):
  del hidden_in, cell_in
  xs = obs[:, 0]
  ys = obs[:, 1]
  occ = _occupancy_sc(xs, ys).reshape(N_AGENTS, N_PAD)
  cells = jnp.arange(N_BINS)
  dest = (cells // N_GRID + 1) * GRID8 + cells % N_GRID + 1
  w64 = jnp.zeros((N_PAD, HIDDEN), jnp.float32).at[dest].set(W.T)
  b2 = b.reshape(1, HIDDEN)
  return _linear_tc(occ, w64, b2)


# speedup vs baseline: 1.2486x; 1.0463x over previous
import functools

import jax
import jax.numpy as jnp
from jax import lax
from jax.experimental import pallas as pl
from jax.experimental.pallas import tpu as pltpu
from jax.experimental.pallas import tpu_sc as plsc

CELL_SIDE = 0.5
N_GRID = 6
N_BINS = N_GRID * N_GRID
GRID8 = 8
N_PAD = GRID8 * GRID8
N_AGENTS = 4096
HIDDEN = 128

NUM_CORES = 2
NUM_SUBCORES = 16
NUM_WORKERS = NUM_CORES * NUM_SUBCORES
ROWS_PER_WORKER = N_AGENTS // NUM_WORKERS
LANES = 16
N_CHUNKS = N_AGENTS // LANES
HIST_WORDS = ROWS_PER_WORKER * N_PAD
SELF_CELL8 = (N_GRID // 2 + 1) * GRID8 + N_GRID // 2 + 1


def _occupancy_sc(xs, ys):
  mesh = plsc.VectorSubcoreMesh(
      core_axis_name="c", subcore_axis_name="s",
      num_cores=NUM_CORES, num_subcores=NUM_SUBCORES)

  @functools.partial(
      pl.kernel,
      out_type=jax.ShapeDtypeStruct((N_AGENTS * N_PAD,), jnp.float32),
      mesh=mesh,
      compiler_params=pltpu.CompilerParams(needs_layout_passes=False),
      scratch_types=[
          pltpu.VMEM((N_AGENTS,), jnp.float32),
          pltpu.VMEM((N_AGENTS,), jnp.float32),
          pltpu.VMEM((HIST_WORDS,), jnp.float32),
      ],
  )
  def occ_kernel(xs_hbm, ys_hbm, out_hbm, xs_v, ys_v, hist_v):
    wid = lax.axis_index("s") * NUM_CORES + lax.axis_index("c")
    base = wid * ROWS_PER_WORKER

    pltpu.sync_copy(xs_hbm, xs_v)
    pltpu.sync_copy(ys_hbm, ys_v)

    inv_cell = 1.0 / CELL_SIDE
    shift = float(N_GRID) / 2.0 + 1.0

    def scale_body(c, _):
      s = pl.ds(c * LANES, LANES)
      xs_v[s] = xs_v[s] * inv_cell
      ys_v[s] = ys_v[s] * inv_cell
      return 0
    lax.fori_loop(0, N_CHUNKS, scale_body, 0)

    zeros16 = jnp.zeros((LANES,), jnp.float32)
    ones16 = jnp.ones((LANES,), jnp.float32)
    top = float(GRID8 - 1)

    def agent_body(ap, _):
      li0 = ap * 2
      li1 = li0 + 1
      i_vec0 = jnp.full((LANES,), base + li0, jnp.int32)
      i_vec1 = jnp.full((LANES,), base + li1, jnp.int32)
      ax0 = plsc.load_gather(xs_v, [i_vec0]) - shift
      ay0 = plsc.load_gather(ys_v, [i_vec0]) - shift
      ax1 = plsc.load_gather(xs_v, [i_vec1]) - shift
      ay1 = plsc.load_gather(ys_v, [i_vec1]) - shift
      row0 = li0 * N_PAD
      row_vec0 = jnp.full((LANES,), row0, jnp.int32)
      row_vec1 = jnp.full((LANES,), li1 * N_PAD, jnp.int32)

      for k in range(2 * N_PAD // LANES):
        hist_v[pl.ds(row0 + k * LANES, LANES)] = zeros16

      @plsc.parallel_loop(0, N_CHUNKS, unroll=2)
      def _(c):
        s = pl.ds(c * LANES, LANES)
        xj = xs_v[s]
        yj = ys_v[s]
        px0 = jnp.clip(xj - ax0, 0.0, top).astype(jnp.int32)
        py0 = jnp.clip(yj - ay0, 0.0, top).astype(jnp.int32)
        plsc.addupdate_scatter(
            hist_v, [row_vec0 + px0 * GRID8 + py0], ones16)
        px1 = jnp.clip(xj - ax1, 0.0, top).astype(jnp.int32)
        py1 = jnp.clip(yj - ay1, 0.0, top).astype(jnp.int32)
        plsc.addupdate_scatter(
            hist_v, [row_vec1 + px1 * GRID8 + py1], ones16)

      return 0

    lax.fori_loop(0, ROWS_PER_WORKER // 2, agent_body, 0)

    pltpu.sync_copy(hist_v, out_hbm.at[pl.ds(base * N_PAD, HIST_WORDS)])

  return occ_kernel(xs, ys)


def _linear_tc(occ, w64, b2):
  def body(occ_ref, wt_ref, b_ref, out_ref):
    acc = jnp.dot(occ_ref[...], wt_ref[...],
                  preferred_element_type=jnp.float32)
    out_ref[...] = acc + (b_ref[...] - wt_ref[SELF_CELL8:SELF_CELL8 + 1, :])

  return pl.pallas_call(
      body,
      out_shape=jax.ShapeDtypeStruct((N_AGENTS, HIDDEN), jnp.float32),
  )(occ, w64, b2)


@jax.jit
def kernel(hidden_in, cell_in, obs, W, b):
  del hidden_in, cell_in
  xs = obs[:, 0]
  ys = obs[:, 1]
  occ = _occupancy_sc(xs, ys).reshape(N_AGENTS, N_PAD)
  cells = jnp.arange(N_BINS)
  dest = (cells // N_GRID + 1) * GRID8 + cells % N_GRID + 1
  w64 = jnp.zeros((N_PAD, HIDDEN), jnp.float32).at[dest].set(W.T)
  b2 = b.reshape(1, HIDDEN)
  return _linear_tc(occ, w64, b2)


# speedup vs baseline: 1.4465x; 1.1585x over previous
import functools

import jax
import jax.numpy as jnp
from jax import lax
from jax.experimental import pallas as pl
from jax.experimental.pallas import tpu as pltpu
from jax.experimental.pallas import tpu_sc as plsc

CELL_SIDE = 0.5
N_GRID = 6
N_BINS = N_GRID * N_GRID
GRID8 = 8
N_PAD = GRID8 * GRID8
N_AGENTS = 4096
HIDDEN = 128

NUM_CORES = 2
NUM_SUBCORES = 16
NUM_WORKERS = NUM_CORES * NUM_SUBCORES
ROWS_PER_WORKER = N_AGENTS // NUM_WORKERS
LANES = 16
N_CHUNKS = N_AGENTS // LANES
HIST_WORDS = ROWS_PER_WORKER * N_PAD
SELF_CELL8 = (N_GRID // 2 + 1) * GRID8 + N_GRID // 2 + 1


def _occupancy_sc(xs, ys):
  mesh = plsc.VectorSubcoreMesh(
      core_axis_name="c", subcore_axis_name="s",
      num_cores=NUM_CORES, num_subcores=NUM_SUBCORES)

  @functools.partial(
      pl.kernel,
      out_type=jax.ShapeDtypeStruct((N_AGENTS * N_PAD,), jnp.float32),
      mesh=mesh,
      compiler_params=pltpu.CompilerParams(needs_layout_passes=False),
      scratch_types=[
          pltpu.VMEM((N_AGENTS,), jnp.float32),
          pltpu.VMEM((N_AGENTS,), jnp.float32),
          pltpu.VMEM((HIST_WORDS,), jnp.float32),
      ],
  )
  def occ_kernel(xs_hbm, ys_hbm, out_hbm, xs_v, ys_v, hist_v):
    wid = lax.axis_index("s") * NUM_CORES + lax.axis_index("c")
    base = wid * ROWS_PER_WORKER

    pltpu.sync_copy(xs_hbm, xs_v)
    pltpu.sync_copy(ys_hbm, ys_v)

    inv_cell = 1.0 / CELL_SIDE
    shift = float(N_GRID) / 2.0 + 1.0

    def scale_body(c, _):
      s = pl.ds(c * LANES, LANES)
      xs_v[s] = xs_v[s] * inv_cell
      ys_v[s] = ys_v[s] * inv_cell
      return 0
    lax.fori_loop(0, N_CHUNKS, scale_body, 0)

    zeros16 = jnp.zeros((LANES,), jnp.float32)
    ones16 = jnp.ones((LANES,), jnp.float32)
    top = float(GRID8 - 1)

    def agent_body(ap, _):
      li0 = ap * 2
      li1 = li0 + 1
      i_vec0 = jnp.full((LANES,), base + li0, jnp.int32)
      i_vec1 = jnp.full((LANES,), base + li1, jnp.int32)
      ax0 = plsc.load_gather(xs_v, [i_vec0]) - shift
      ay0 = plsc.load_gather(ys_v, [i_vec0]) - shift
      ax1 = plsc.load_gather(xs_v, [i_vec1]) - shift
      ay1 = plsc.load_gather(ys_v, [i_vec1]) - shift
      row0 = li0 * N_PAD
      row_vec0 = jnp.full((LANES,), row0, jnp.int32)
      row_vec1 = jnp.full((LANES,), li1 * N_PAD, jnp.int32)

      for k in range(2 * N_PAD // LANES):
        hist_v[pl.ds(row0 + k * LANES, LANES)] = zeros16

      @plsc.parallel_loop(0, N_CHUNKS, unroll=4)
      def _(c):
        s = pl.ds(c * LANES, LANES)
        xj = xs_v[s]
        yj = ys_v[s]
        px0 = jnp.clip(xj - ax0, 0.0, top).astype(jnp.int32)
        py0 = jnp.clip(yj - ay0, 0.0, top).astype(jnp.int32)
        plsc.addupdate_scatter(
            hist_v, [row_vec0 + px0 * GRID8 + py0], ones16)
        px1 = jnp.clip(xj - ax1, 0.0, top).astype(jnp.int32)
        py1 = jnp.clip(yj - ay1, 0.0, top).astype(jnp.int32)
        plsc.addupdate_scatter(
            hist_v, [row_vec1 + px1 * GRID8 + py1], ones16)

      return 0

    lax.fori_loop(0, ROWS_PER_WORKER // 2, agent_body, 0)

    pltpu.sync_copy(hist_v, out_hbm.at[pl.ds(base * N_PAD, HIST_WORDS)])

  return occ_kernel(xs, ys)


def _linear_tc(occ, w64, b2):
  def body(occ_ref, wt_ref, b_ref, out_ref):
    acc = jnp.dot(occ_ref[...], wt_ref[...],
                  preferred_element_type=jnp.float32)
    out_ref[...] = acc + (b_ref[...] - wt_ref[SELF_CELL8:SELF_CELL8 + 1, :])

  return pl.pallas_call(
      body,
      out_shape=jax.ShapeDtypeStruct((N_AGENTS, HIDDEN), jnp.float32),
  )(occ, w64, b2)


@jax.jit
def kernel(hidden_in, cell_in, obs, W, b):
  del hidden_in, cell_in
  xs = obs[:, 0]
  ys = obs[:, 1]
  occ = _occupancy_sc(xs, ys).reshape(N_AGENTS, N_PAD)
  cells = jnp.arange(N_BINS)
  dest = (cells // N_GRID + 1) * GRID8 + cells % N_GRID + 1
  w64 = jnp.zeros((N_PAD, HIDDEN), jnp.float32).at[dest].set(W.T)
  b2 = b.reshape(1, HIDDEN)
  return _linear_tc(occ, w64, b2)
